# Initial kernel scaffold; baseline (speedup 1.0000x reference)
import jax
import jax.numpy as jnp
from jax.experimental import pallas as pl


def kernel(xs, weights):
    raise NotImplementedError("write your pallas kernel here")


# speedup vs baseline: 4.0825x; 4.0825x over previous
import functools

import jax
import jax.numpy as jnp
from jax import lax
from jax.experimental import pallas as pl
from jax.experimental.pallas import tpu as pltpu
from jax.experimental.pallas import tpu_sc as plsc

_NUM_CORES = 2
_NUM_SUBCORES = 16
_NW = _NUM_CORES * _NUM_SUBCORES
_CHUNK = 128


@functools.lru_cache(maxsize=None)
def _build_gather(n, v, d):
    per_w = n // _NW
    nchunk = per_w // _CHUNK
    assert per_w % _CHUNK == 0

    mesh = plsc.VectorSubcoreMesh(core_axis_name="c", subcore_axis_name="s")

    @functools.partial(
        pl.kernel,
        mesh=mesh,
        out_type=jax.ShapeDtypeStruct((n, d), jnp.float32),
        compiler_params=pltpu.CompilerParams(use_tc_tiling_on_sc=False),
        scratch_types=[
            pltpu.VMEM((nchunk, _CHUNK), jnp.int32),
            pltpu.VMEM((_CHUNK, d), jnp.float32),
            pltpu.SemaphoreType.DMA,
        ],
    )
    def gather_kernel(table_hbm, idx_hbm, out_hbm, idx_v, rows_v, sem):
        wid = lax.axis_index("s") * _NUM_CORES + lax.axis_index("c")
        base = wid * per_w
        pltpu.sync_copy(idx_hbm.at[wid], idx_v)

        def body(j, carry):
            pltpu.async_copy(table_hbm.at[idx_v.at[j]], rows_v, sem).wait()
            pltpu.sync_copy(
                rows_v, out_hbm.at[pl.ds(base + j * _CHUNK, _CHUNK)]
            )
            return carry

        lax.fori_loop(0, nchunk, body, 0)

    return gather_kernel


def kernel(xs, weights):
    b, h = xs.shape
    v, d = weights.shape
    n = b * h
    idx = xs.reshape(_NW, n // (_NW * _CHUNK), _CHUNK).astype(jnp.int32)
    out = _build_gather(n, v, d)(weights, idx)
    return out.reshape(b, h, d)


# speedup vs baseline: 4.5953x; 1.1256x over previous
import functools

import jax
import jax.numpy as jnp
from jax import lax
from jax.experimental import pallas as pl
from jax.experimental.pallas import tpu as pltpu
from jax.experimental.pallas import tpu_sc as plsc

_NUM_CORES = 2
_NUM_SUBCORES = 16
_NW = _NUM_CORES * _NUM_SUBCORES
_CHUNK = 640


@functools.lru_cache(maxsize=None)
def _build_gather(n, v, d):
    per_w = n // _NW
    nchunk = per_w // _CHUNK
    assert per_w % _CHUNK == 0 and nchunk % 2 == 0

    mesh = plsc.VectorSubcoreMesh(core_axis_name="c", subcore_axis_name="s")

    @functools.partial(
        pl.kernel,
        mesh=mesh,
        out_type=jax.ShapeDtypeStruct((n, d), jnp.float32),
        compiler_params=pltpu.CompilerParams(use_tc_tiling_on_sc=False),
        scratch_types=[
            pltpu.VMEM((per_w,), jnp.int32),
            pltpu.VMEM((2, _CHUNK, d), jnp.float32),
            pltpu.SemaphoreType.DMA,
            pltpu.SemaphoreType.DMA,
            pltpu.SemaphoreType.DMA,
            pltpu.SemaphoreType.DMA,
        ],
    )
    def gather_kernel(table_hbm, idx_hbm, out_hbm, idx_v, rows_v, g0, g1, w0, w1):
        wid = lax.axis_index("s") * _NUM_CORES + lax.axis_index("c")
        base = wid * per_w
        pltpu.sync_copy(idx_hbm.at[wid], idx_v)

        def chunk_ops(j, b, gsem, wsem):
            gather = pltpu.make_async_copy(
                table_hbm.at[idx_v.at[pl.ds(j * _CHUNK, _CHUNK)]],
                rows_v.at[b],
                gsem,
            )
            write = pltpu.make_async_copy(
                rows_v.at[b],
                out_hbm.at[pl.ds(base + j * _CHUNK, _CHUNK)],
                wsem,
            )
            return gather, write

        def body(go, carry):
            j0 = 2 * go
            g_a, w_a = chunk_ops(j0, 0, g0, w0)
            g_b, w_b = chunk_ops(j0 + 1, 1, g1, w1)

            @pl.when(go > 0)
            def _():
                w_a.wait()

            g_a.start()

            @pl.when(go > 0)
            def _():
                w_b.wait()

            g_b.start()

            g_a.wait()
            w_a.start()
            g_b.wait()
            w_b.start()
            return carry

        lax.fori_loop(0, nchunk // 2, body, 0)

        _, w_a = chunk_ops(nchunk - 2, 0, g0, w0)
        _, w_b = chunk_ops(nchunk - 1, 1, g1, w1)
        w_a.wait()
        w_b.wait()

    return gather_kernel


def kernel(xs, weights):
    b, h = xs.shape
    v, d = weights.shape
    n = b * h
    idx = xs.reshape(_NW, n // _NW).astype(jnp.int32)
    out = _build_gather(n, v, d)(weights, idx)
    return out.reshape(b, h, d)
